# baseline (speedup 1.0000x reference)
import jax
import jax.numpy as jnp
from jax import lax
from jax.experimental import pallas as pl
from jax.experimental.pallas import tpu as pltpu
from jax.experimental.pallas import tpu_sc as plsc

f32 = jnp.float32
i32 = jnp.int32

N = 10000
NP = 10240
DUMMY = N
E = 320000
D = 128
CH = 128
CPT = 160
NBUF = 4
NTILES = 16
CHUNKS = NTILES * CPT
EP = CHUNKS * CH
RPT = NP // NTILES

DE = (94, 72, 50, 16)
BT = 1024


def _sc_mesh():
    return plsc.VectorSubcoreMesh(
        core_axis_name="c", subcore_axis_name="s", num_cores=2, num_subcores=16)


def _deg_body(dstc, out, idx_v, acc_v):
    c = lax.axis_index("c")
    s = lax.axis_index("s")
    w = c * NTILES + s
    zero16 = jnp.zeros((16,), f32)
    ones16 = jnp.ones((16,), f32)

    @pl.loop(0, NP // 16)
    def _(i):
        acc_v[pl.ds(i * 16, 16)] = zero16

    dpt = CHUNKS // 32
    pltpu.sync_copy(dstc.at[pl.ds(w * dpt, dpt)], idx_v)

    @pl.loop(0, dpt)
    def _(j):
        row = idx_v.at[j]
        for q in range(8):
            idx16 = row[pl.ds(q * 16, 16)]
            plsc.addupdate_scatter(acc_v, [idx16], ones16)

    @pl.when(c == 0)
    def _():
        pltpu.sync_copy(acc_v, out.at[0].at[s])

    @pl.when(c == 1)
    def _():
        pltpu.sync_copy(acc_v, out.at[1].at[s])


_deg_kernel = pl.kernel(
    _deg_body,
    out_type=jax.ShapeDtypeStruct((2, NTILES, NP), f32),
    mesh=_sc_mesh(),
    compiler_params=pltpu.CompilerParams(needs_layout_passes=False),
    scratch_types=[
        pltpu.VMEM((CHUNKS // 32, CH), i32),
        pltpu.VMEM((NP,), f32),
    ],
)


NB = 4
BR = NP // NB
RW = 32
RCH = CHUNKS // RW
EPT = RCH * CH
BRA = BR + 128
ZPT = BR // NTILES


def _bucket_body(srcc, dstc, bsrc, bdst, cnts, sidx_v, didx_v, *bufs):
    c = lax.axis_index("c")
    s = lax.axis_index("s")
    w = c * NTILES + s
    sbufs = bufs[:NB]
    dbufs = bufs[NB:2 * NB]
    cbuf_v = bufs[2 * NB]
    pltpu.sync_copy(srcc.at[pl.ds(w * RCH, RCH)], sidx_v)
    pltpu.sync_copy(dstc.at[pl.ds(w * RCH, RCH)], didx_v)

    dummy16 = jnp.full((16,), DUMMY, i32)
    lane16 = lax.iota(i32, 16)

    @pl.loop(0, EPT // 16)
    def _(i):
        trash16 = BR + lax.rem(i, 8) * 16 + lane16
        for b in range(NB):
            sbufs[b][pl.ds(i * 16, 16)] = dummy16
            dbufs[b][pl.ds(i * 16, 16)] = trash16

    def chunk(j, offs):
        srow = sidx_v.at[j]
        drow = didx_v.at[j]
        for q in range(8):
            sv = srow[pl.ds(q * 16, 16)]
            dv = drow[pl.ds(q * 16, 16)]
            new = []
            for b in range(NB):
                m = (dv >= b * BR) & (dv < (b + 1) * BR)
                cs = plsc.cumsum(jnp.where(m, 1, 0).astype(i32))
                pos = offs[b] + cs - 1
                plsc.store_scatter(sbufs[b], [pos], sv, mask=m)
                plsc.store_scatter(dbufs[b], [pos], dv - b * BR, mask=m)
                new.append(offs[b] + jnp.max(cs))
            offs = tuple(new)
        return offs

    z = jnp.int32(0)
    offs = lax.fori_loop(0, RCH, chunk, (z,) * NB)
    cv = jnp.zeros((16,), i32)
    lanes = lax.iota(i32, 16)
    for b in range(NB):
        cv = jnp.where(lanes == b, offs[b], cv)
    cbuf_v[...] = cv
    for b in range(NB):
        pltpu.sync_copy(sbufs[b], bsrc.at[w, b])
        pltpu.sync_copy(dbufs[b], bdst.at[w, b])
    pltpu.sync_copy(cbuf_v, cnts.at[w])


_bucket_kernel = pl.kernel(
    _bucket_body,
    out_type=(
        jax.ShapeDtypeStruct((RW, NB, EPT), i32),
        jax.ShapeDtypeStruct((RW, NB, EPT), i32),
        jax.ShapeDtypeStruct((RW, 16), i32),
    ),
    mesh=_sc_mesh(),
    compiler_params=pltpu.CompilerParams(needs_layout_passes=False),
    scratch_types=(
        [pltpu.VMEM((RCH, CH), i32), pltpu.VMEM((RCH, CH), i32)]
        + [pltpu.VMEM((EPT,), i32) for _ in range(2 * NB)]
        + [pltpu.VMEM((16,), i32)]
    ),
)


KPF = 2
SLOTS = 2 * KPF
RCHP = RCH


def _prop_body(tab, bsrc, bdst, cnts, zrows, out, sidx_v, didx_v, didx2_v,
               rows_v, cnt_vm, acc_sh, sem_g, sem_s):
    c = lax.axis_index("c")
    s = lax.axis_index("s")
    pltpu.sync_copy(cnts, cnt_vm)
    lanes = lax.iota(i32, 16)

    def run(tab_h, out_h):
        for b in range(NB):
            pltpu.sync_copy(zrows.at[pl.ds(s * ZPT, ZPT)],
                            acc_sh.at[pl.ds(s * ZPT, ZPT)])
            plsc.subcore_barrier()
            for r in range(2):
                w = s * 2 + r
                pltpu.sync_copy(bsrc.at[w, b], sidx_v)
                pltpu.sync_copy(bdst.at[w, b], didx_v)
                cv = cnt_vm[w]
                cnt = jnp.max(jnp.where(lanes == b, cv, 0))
                nchr = (lax.div(lax.div(cnt + (CH - 1), CH) + (SLOTS - 1),
                                SLOTS) * SLOTS)

                def gather(j, t):
                    pltpu.async_copy(
                        tab_h.at[sidx_v.at[pl.ds(j * CH, CH)]],
                        rows_v.at[t], sem_g.at[t])

                def drain_s(t):
                    pltpu.make_async_copy(
                        tab_h.at[pl.ds(0, CH)], rows_v.at[t],
                        sem_s.at[t]).wait()

                for t in range(KPF):
                    @pl.when(t < nchr)
                    def _():
                        gather(t, t)

                def group(g, _):
                    for t in range(SLOTS):
                        j = g * SLOTS + t
                        pltpu.make_async_copy(
                            tab_h.at[pl.ds(0, CH)], rows_v.at[t],
                            sem_g.at[t]).wait()
                        for q in range(CH // 16):
                            didx2_v[t, pl.ds(q * 16, 16)] = (
                                didx_v[pl.ds(j * CH + q * 16, 16)])
                        t2 = (t + KPF) % SLOTS

                        @pl.when(j + KPF < nchr)
                        def _():
                            @pl.when(j + KPF >= SLOTS)
                            def _():
                                drain_s(t2)
                            gather(j + KPF, t2)
                        pltpu.async_copy(rows_v.at[t],
                                         acc_sh.at[didx2_v.at[t]],
                                         sem_s.at[t], add=True)
                    return 0

                lax.fori_loop(0, lax.div(nchr, SLOTS), group, 0)

                @pl.when(nchr > 0)
                def _():
                    for t in range(SLOTS):
                        drain_s(t)
            plsc.subcore_barrier()
            pltpu.sync_copy(acc_sh.at[pl.ds(s * ZPT, ZPT)],
                            out_h.at[pl.ds(b * BR + s * ZPT, ZPT)])
            plsc.subcore_barrier()

    @pl.when(c == 0)
    def _():
        run(tab.at[0], out.at[0])

    @pl.when(c == 1)
    def _():
        run(tab.at[1], out.at[1])


_prop_kernel = pl.kernel(
    _prop_body,
    out_type=jax.ShapeDtypeStruct((2, NP, D), f32),
    mesh=_sc_mesh(),
    compiler_params=pltpu.CompilerParams(needs_layout_passes=False),
    scratch_types=[
        pltpu.VMEM((EPT,), i32),
        pltpu.VMEM((EPT,), i32),
        pltpu.VMEM((SLOTS, CH), i32),
        pltpu.VMEM((SLOTS, CH, D), f32),
        pltpu.VMEM((RW, 16), i32),
        pltpu.VMEM_SHARED((BRA, D), f32),
        pltpu.SemaphoreType.DMA((SLOTS,)),
        pltpu.SemaphoreType.DMA((SLOTS,)),
    ],
)


def _round0_body(x_ref, deg_ref, wn_ref, we_ref, out_ref, dinv_ref):
    i = pl.program_id(0)
    deg = jnp.sum(deg_ref[...], axis=(0, 1)) + 1.0
    rows = i * BT + lax.broadcasted_iota(i32, (BT,), 0)
    dinv = jnp.where(rows < N, lax.rsqrt(deg), 0.0)
    dinv_ref[...] = dinv
    xv = x_ref[...]
    un = jnp.dot(xv, wn_ref[...], preferred_element_type=f32,
                 precision=lax.Precision.HIGHEST)
    ue = jnp.dot(xv, we_ref[...], preferred_element_type=f32,
                 precision=lax.Precision.HIGHEST)
    out_ref[0] = dinv[:, None] * un
    out_ref[1] = dinv[:, None] * ue


_round0 = pl.pallas_call(
    _round0_body,
    grid=(NP // BT,),
    in_specs=[
        pl.BlockSpec((BT, D), lambda i: (i, 0)),
        pl.BlockSpec((2, NTILES, BT), lambda i: (0, 0, i)),
        pl.BlockSpec((D, D), lambda i: (0, 0)),
        pl.BlockSpec((D, D), lambda i: (0, 0)),
    ],
    out_specs=[
        pl.BlockSpec((2, BT, D), lambda i: (0, i, 0)),
        pl.BlockSpec((BT,), lambda i: (i,)),
    ],
    out_shape=[
        jax.ShapeDtypeStruct((2, NP, D), f32),
        jax.ShapeDtypeStruct((NP,), f32),
    ],
)


def _round_body(acc_ref, tab_ref, dinv_ref, wn_ref, we_ref, b_ref, out_ref):
    dinv = dinv_ref[...][:, None]
    pre_n = dinv * (acc_ref[0] + tab_ref[0]) + b_ref[0]
    pre_e = dinv * (acc_ref[1] + tab_ref[1]) + b_ref[1]
    act_n = jnp.maximum(pre_n, 0.0)
    act_e = jnp.maximum(pre_e, 0.0)
    un = jnp.dot(act_n, wn_ref[...], preferred_element_type=f32,
                 precision=lax.Precision.HIGHEST)
    ue = jnp.dot(act_e, we_ref[...], preferred_element_type=f32,
                 precision=lax.Precision.HIGHEST)
    out_ref[0] = dinv * un
    out_ref[1] = dinv * ue


_round = pl.pallas_call(
    _round_body,
    grid=(NP // BT,),
    in_specs=[
        pl.BlockSpec((2, BT, D), lambda i: (0, i, 0)),
        pl.BlockSpec((2, BT, D), lambda i: (0, i, 0)),
        pl.BlockSpec((BT,), lambda i: (i,)),
        pl.BlockSpec((D, D), lambda i: (0, 0)),
        pl.BlockSpec((D, D), lambda i: (0, 0)),
        pl.BlockSpec((2, 1, D), lambda i: (0, 0, 0)),
    ],
    out_specs=pl.BlockSpec((2, BT, D), lambda i: (0, i, 0)),
    out_shape=jax.ShapeDtypeStruct((2, NP, D), f32),
)


def _final_body(acc_ref, tab_ref, dinv_ref, b_ref, outn_ref, oute_ref):
    dinv = dinv_ref[...][:, None]
    pre_n = dinv * (acc_ref[0] + tab_ref[0]) + b_ref[0]
    pre_e = dinv * (acc_ref[1] + tab_ref[1]) + b_ref[1]
    outn_ref[...] = jax.nn.sigmoid(pre_n)
    oute_ref[...] = jax.nn.sigmoid(pre_e[:, :16])


_final = pl.pallas_call(
    _final_body,
    grid=(NP // BT,),
    in_specs=[
        pl.BlockSpec((2, BT, D), lambda i: (0, i, 0)),
        pl.BlockSpec((2, BT, D), lambda i: (0, i, 0)),
        pl.BlockSpec((BT,), lambda i: (i,)),
        pl.BlockSpec((2, 1, D), lambda i: (0, 0, 0)),
    ],
    out_specs=[
        pl.BlockSpec((BT, D), lambda i: (i, 0)),
        pl.BlockSpec((BT, 16), lambda i: (i, 0)),
    ],
    out_shape=[
        jax.ShapeDtypeStruct((NP, D), f32),
        jax.ShapeDtypeStruct((NP, 16), f32),
    ],
)


def _pad_w(W):
    din, dout = W.shape
    return jnp.zeros((D, D), f32).at[:din, :dout].set(W)


def _pad_b2(bn, be):
    b = jnp.zeros((2, 1, D), f32)
    return b.at[0, 0, :].set(bn).at[1, 0, :be.shape[0]].set(be)


def kernel(x, edge_index, W1e, b1e, W2e, b2e, W3e, b3e, W4e, b4e,
           W1n, b1n, W2n, b2n, W3n, b3n, W4n, b4n):
    src = edge_index[0].astype(i32)
    dst = edge_index[1].astype(i32)
    pad = jnp.full((EP - E,), DUMMY, i32)
    srcc = jnp.concatenate([src, pad]).reshape(CHUNKS, CH)
    dstc = jnp.concatenate([dst, pad]).reshape(CHUNKS, CH)
    xp = jnp.pad(x, ((0, NP - N), (0, 0)))
    zrows = jnp.zeros((NP, D), f32)

    wn = [_pad_w(W1n), _pad_w(W2n), _pad_w(W3n), _pad_w(W4n)]
    we = [_pad_w(W1e), _pad_w(W2e), _pad_w(W3e), _pad_w(W4e)]
    bb = [_pad_b2(b1n, b1e), _pad_b2(b2n, b2e), _pad_b2(b3n, b3e),
          _pad_b2(b4n, b4e)]

    deg_parts = _deg_kernel(dstc)
    bsrc, bdst, cnts = _bucket_kernel(srcc, dstc)
    t, dinv = _round0(xp, deg_parts, wn[0], we[0])
    a = _prop_kernel(t, bsrc, bdst, cnts, zrows)
    t = _round(a, t, dinv, wn[1], we[1], bb[0])
    a = _prop_kernel(t, bsrc, bdst, cnts, zrows)
    t = _round(a, t, dinv, wn[2], we[2], bb[1])
    a = _prop_kernel(t, bsrc, bdst, cnts, zrows)
    t = _round(a, t, dinv, wn[3], we[3], bb[2])
    a = _prop_kernel(t, bsrc, bdst, cnts, zrows)
    nodes_p, edges_p = _final(a, t, dinv, bb[3])

    return edges_p[:N], nodes_p[:N]


# speedup vs baseline: 1.0069x; 1.0069x over previous
import jax
import jax.numpy as jnp
from jax import lax
from jax.experimental import pallas as pl
from jax.experimental.pallas import tpu as pltpu
from jax.experimental.pallas import tpu_sc as plsc

f32 = jnp.float32
i32 = jnp.int32

N = 10000
NP = 10240
DUMMY = N
E = 320000
D = 128
CH = 128
CPT = 160
NBUF = 4
NTILES = 16
CHUNKS = NTILES * CPT
EP = CHUNKS * CH
RPT = NP // NTILES

DE = (94, 72, 50, 16)
BT = 1024


def _sc_mesh():
    return plsc.VectorSubcoreMesh(
        core_axis_name="c", subcore_axis_name="s", num_cores=2, num_subcores=16)


def _deg_body(dstc, out, idx_v, acc_v):
    c = lax.axis_index("c")
    s = lax.axis_index("s")
    w = c * NTILES + s
    zero16 = jnp.zeros((16,), f32)
    ones16 = jnp.ones((16,), f32)

    @pl.loop(0, NP // 16)
    def _(i):
        acc_v[pl.ds(i * 16, 16)] = zero16

    dpt = CHUNKS // 32
    pltpu.sync_copy(dstc.at[pl.ds(w * dpt, dpt)], idx_v)

    @pl.loop(0, dpt)
    def _(j):
        row = idx_v.at[j]
        for q in range(8):
            idx16 = row[pl.ds(q * 16, 16)]
            plsc.addupdate_scatter(acc_v, [idx16], ones16)

    @pl.when(c == 0)
    def _():
        pltpu.sync_copy(acc_v, out.at[0].at[s])

    @pl.when(c == 1)
    def _():
        pltpu.sync_copy(acc_v, out.at[1].at[s])


_deg_kernel = pl.kernel(
    _deg_body,
    out_type=jax.ShapeDtypeStruct((2, NTILES, NP), f32),
    mesh=_sc_mesh(),
    compiler_params=pltpu.CompilerParams(needs_layout_passes=False),
    scratch_types=[
        pltpu.VMEM((CHUNKS // 32, CH), i32),
        pltpu.VMEM((NP,), f32),
    ],
)


NB = 4
BR = NP // NB
RW = 32
RCH = CHUNKS // RW
EPT = RCH * CH
BRA = BR + 128
ZPT = BR // NTILES


def _bucket_body(srcc, dstc, bsrc, bdst, cnts, sidx_v, didx_v, *bufs):
    c = lax.axis_index("c")
    s = lax.axis_index("s")
    w = c * NTILES + s
    sbufs = bufs[:NB]
    dbufs = bufs[NB:2 * NB]
    cbuf_v = bufs[2 * NB]
    pltpu.sync_copy(srcc.at[pl.ds(w * RCH, RCH)], sidx_v)
    pltpu.sync_copy(dstc.at[pl.ds(w * RCH, RCH)], didx_v)

    dummy16 = jnp.full((16,), DUMMY, i32)
    lane16 = lax.iota(i32, 16)

    @pl.loop(0, EPT // 16)
    def _(i):
        trash16 = BR + lax.rem(i, 8) * 16 + lane16
        for b in range(NB):
            sbufs[b][pl.ds(i * 16, 16)] = dummy16
            dbufs[b][pl.ds(i * 16, 16)] = trash16

    def chunk(j, offs):
        srow = sidx_v.at[j]
        drow = didx_v.at[j]
        for q in range(8):
            sv = srow[pl.ds(q * 16, 16)]
            dv = drow[pl.ds(q * 16, 16)]
            new = []
            for b in range(NB):
                m = (dv >= b * BR) & (dv < (b + 1) * BR)
                cs = plsc.cumsum(jnp.where(m, 1, 0).astype(i32))
                pos = offs[b] + cs - 1
                plsc.store_scatter(sbufs[b], [pos], sv, mask=m)
                plsc.store_scatter(dbufs[b], [pos], dv - b * BR, mask=m)
                new.append(offs[b] + jnp.max(cs))
            offs = tuple(new)
        return offs

    z = jnp.int32(0)
    offs = lax.fori_loop(0, RCH, chunk, (z,) * NB)
    cv = jnp.zeros((16,), i32)
    lanes = lax.iota(i32, 16)
    for b in range(NB):
        cv = jnp.where(lanes == b, offs[b], cv)
    cbuf_v[...] = cv
    for b in range(NB):
        pltpu.sync_copy(sbufs[b], bsrc.at[w, b])
        pltpu.sync_copy(dbufs[b], bdst.at[w, b])
    pltpu.sync_copy(cbuf_v, cnts.at[w])


_bucket_kernel = pl.kernel(
    _bucket_body,
    out_type=(
        jax.ShapeDtypeStruct((RW, NB, EPT), i32),
        jax.ShapeDtypeStruct((RW, NB, EPT), i32),
        jax.ShapeDtypeStruct((RW, 16), i32),
    ),
    mesh=_sc_mesh(),
    compiler_params=pltpu.CompilerParams(needs_layout_passes=False),
    scratch_types=(
        [pltpu.VMEM((RCH, CH), i32), pltpu.VMEM((RCH, CH), i32)]
        + [pltpu.VMEM((EPT,), i32) for _ in range(2 * NB)]
        + [pltpu.VMEM((16,), i32)]
    ),
)


KPF = 3
SLOTS = 4
RCHP = RCH


def _prop_body(tab, bsrc, bdst, cnts, zrows, out, sidx_v, didx_v, didx2_v,
               rows_v, cnt_vm, acc_sh, sem_g, sem_s):
    c = lax.axis_index("c")
    s = lax.axis_index("s")
    pltpu.sync_copy(cnts, cnt_vm)
    lanes = lax.iota(i32, 16)

    def run(tab_h, out_h):
        for b in range(NB):
            pltpu.sync_copy(zrows.at[pl.ds(s * ZPT, ZPT)],
                            acc_sh.at[pl.ds(s * ZPT, ZPT)])
            plsc.subcore_barrier()
            for r in range(2):
                w = s * 2 + r
                pltpu.sync_copy(bsrc.at[w, b], sidx_v)
                pltpu.sync_copy(bdst.at[w, b], didx_v)
                cv = cnt_vm[w]
                cnt = jnp.max(jnp.where(lanes == b, cv, 0))
                nchr = (lax.div(lax.div(cnt + (CH - 1), CH) + (SLOTS - 1),
                                SLOTS) * SLOTS)

                def gather(j, t):
                    pltpu.async_copy(
                        tab_h.at[sidx_v.at[pl.ds(j * CH, CH)]],
                        rows_v.at[t], sem_g.at[t])

                for t in range(KPF):
                    @pl.when(t < nchr)
                    def _():
                        gather(t, t)

                def group(g, _):
                    for t in range(SLOTS):
                        j = g * SLOTS + t
                        pltpu.make_async_copy(
                            tab_h.at[pl.ds(0, CH)], rows_v.at[t],
                            sem_g.at[t]).wait()
                        for q in range(CH // 16):
                            didx2_v[t, pl.ds(q * 16, 16)] = (
                                didx_v[pl.ds(j * CH + q * 16, 16)])
                        t2 = (t + KPF) % SLOTS

                        @pl.when(j + KPF < nchr)
                        def _():
                            gather(j + KPF, t2)
                        pltpu.sync_copy(rows_v.at[t],
                                        acc_sh.at[didx2_v.at[t]],
                                        add=True)
                    return 0

                lax.fori_loop(0, lax.div(nchr, SLOTS), group, 0)
            plsc.subcore_barrier()
            pltpu.sync_copy(acc_sh.at[pl.ds(s * ZPT, ZPT)],
                            out_h.at[pl.ds(b * BR + s * ZPT, ZPT)])
            plsc.subcore_barrier()

    @pl.when(c == 0)
    def _():
        run(tab.at[0], out.at[0])

    @pl.when(c == 1)
    def _():
        run(tab.at[1], out.at[1])


_prop_kernel = pl.kernel(
    _prop_body,
    out_type=jax.ShapeDtypeStruct((2, NP, D), f32),
    mesh=_sc_mesh(),
    compiler_params=pltpu.CompilerParams(needs_layout_passes=False),
    scratch_types=[
        pltpu.VMEM((EPT,), i32),
        pltpu.VMEM((EPT,), i32),
        pltpu.VMEM((SLOTS, CH), i32),
        pltpu.VMEM((SLOTS, CH, D), f32),
        pltpu.VMEM((RW, 16), i32),
        pltpu.VMEM_SHARED((BRA, D), f32),
        pltpu.SemaphoreType.DMA((SLOTS,)),
        pltpu.SemaphoreType.DMA((SLOTS,)),
    ],
)


def _round0_body(x_ref, deg_ref, wn_ref, we_ref, out_ref, dinv_ref):
    i = pl.program_id(0)
    deg = jnp.sum(deg_ref[...], axis=(0, 1)) + 1.0
    rows = i * BT + lax.broadcasted_iota(i32, (BT,), 0)
    dinv = jnp.where(rows < N, lax.rsqrt(deg), 0.0)
    dinv_ref[...] = dinv
    xv = x_ref[...]
    un = jnp.dot(xv, wn_ref[...], preferred_element_type=f32,
                 precision=lax.Precision.HIGHEST)
    ue = jnp.dot(xv, we_ref[...], preferred_element_type=f32,
                 precision=lax.Precision.HIGHEST)
    out_ref[0] = dinv[:, None] * un
    out_ref[1] = dinv[:, None] * ue


_round0 = pl.pallas_call(
    _round0_body,
    grid=(NP // BT,),
    in_specs=[
        pl.BlockSpec((BT, D), lambda i: (i, 0)),
        pl.BlockSpec((2, NTILES, BT), lambda i: (0, 0, i)),
        pl.BlockSpec((D, D), lambda i: (0, 0)),
        pl.BlockSpec((D, D), lambda i: (0, 0)),
    ],
    out_specs=[
        pl.BlockSpec((2, BT, D), lambda i: (0, i, 0)),
        pl.BlockSpec((BT,), lambda i: (i,)),
    ],
    out_shape=[
        jax.ShapeDtypeStruct((2, NP, D), f32),
        jax.ShapeDtypeStruct((NP,), f32),
    ],
)


def _round_body(acc_ref, tab_ref, dinv_ref, wn_ref, we_ref, b_ref, out_ref):
    dinv = dinv_ref[...][:, None]
    pre_n = dinv * (acc_ref[0] + tab_ref[0]) + b_ref[0]
    pre_e = dinv * (acc_ref[1] + tab_ref[1]) + b_ref[1]
    act_n = jnp.maximum(pre_n, 0.0)
    act_e = jnp.maximum(pre_e, 0.0)
    un = jnp.dot(act_n, wn_ref[...], preferred_element_type=f32,
                 precision=lax.Precision.HIGHEST)
    ue = jnp.dot(act_e, we_ref[...], preferred_element_type=f32,
                 precision=lax.Precision.HIGHEST)
    out_ref[0] = dinv * un
    out_ref[1] = dinv * ue


_round = pl.pallas_call(
    _round_body,
    grid=(NP // BT,),
    in_specs=[
        pl.BlockSpec((2, BT, D), lambda i: (0, i, 0)),
        pl.BlockSpec((2, BT, D), lambda i: (0, i, 0)),
        pl.BlockSpec((BT,), lambda i: (i,)),
        pl.BlockSpec((D, D), lambda i: (0, 0)),
        pl.BlockSpec((D, D), lambda i: (0, 0)),
        pl.BlockSpec((2, 1, D), lambda i: (0, 0, 0)),
    ],
    out_specs=pl.BlockSpec((2, BT, D), lambda i: (0, i, 0)),
    out_shape=jax.ShapeDtypeStruct((2, NP, D), f32),
)


def _final_body(acc_ref, tab_ref, dinv_ref, b_ref, outn_ref, oute_ref):
    dinv = dinv_ref[...][:, None]
    pre_n = dinv * (acc_ref[0] + tab_ref[0]) + b_ref[0]
    pre_e = dinv * (acc_ref[1] + tab_ref[1]) + b_ref[1]
    outn_ref[...] = jax.nn.sigmoid(pre_n)
    oute_ref[...] = jax.nn.sigmoid(pre_e[:, :16])


_final = pl.pallas_call(
    _final_body,
    grid=(NP // BT,),
    in_specs=[
        pl.BlockSpec((2, BT, D), lambda i: (0, i, 0)),
        pl.BlockSpec((2, BT, D), lambda i: (0, i, 0)),
        pl.BlockSpec((BT,), lambda i: (i,)),
        pl.BlockSpec((2, 1, D), lambda i: (0, 0, 0)),
    ],
    out_specs=[
        pl.BlockSpec((BT, D), lambda i: (i, 0)),
        pl.BlockSpec((BT, 16), lambda i: (i, 0)),
    ],
    out_shape=[
        jax.ShapeDtypeStruct((NP, D), f32),
        jax.ShapeDtypeStruct((NP, 16), f32),
    ],
)


def _pad_w(W):
    din, dout = W.shape
    return jnp.zeros((D, D), f32).at[:din, :dout].set(W)


def _pad_b2(bn, be):
    b = jnp.zeros((2, 1, D), f32)
    return b.at[0, 0, :].set(bn).at[1, 0, :be.shape[0]].set(be)


def kernel(x, edge_index, W1e, b1e, W2e, b2e, W3e, b3e, W4e, b4e,
           W1n, b1n, W2n, b2n, W3n, b3n, W4n, b4n):
    src = edge_index[0].astype(i32)
    dst = edge_index[1].astype(i32)
    pad = jnp.full((EP - E,), DUMMY, i32)
    srcc = jnp.concatenate([src, pad]).reshape(CHUNKS, CH)
    dstc = jnp.concatenate([dst, pad]).reshape(CHUNKS, CH)
    xp = jnp.pad(x, ((0, NP - N), (0, 0)))
    zrows = jnp.zeros((NP, D), f32)

    wn = [_pad_w(W1n), _pad_w(W2n), _pad_w(W3n), _pad_w(W4n)]
    we = [_pad_w(W1e), _pad_w(W2e), _pad_w(W3e), _pad_w(W4e)]
    bb = [_pad_b2(b1n, b1e), _pad_b2(b2n, b2e), _pad_b2(b3n, b3e),
          _pad_b2(b4n, b4e)]

    deg_parts = _deg_kernel(dstc)
    bsrc, bdst, cnts = _bucket_kernel(srcc, dstc)
    t, dinv = _round0(xp, deg_parts, wn[0], we[0])
    a = _prop_kernel(t, bsrc, bdst, cnts, zrows)
    t = _round(a, t, dinv, wn[1], we[1], bb[0])
    a = _prop_kernel(t, bsrc, bdst, cnts, zrows)
    t = _round(a, t, dinv, wn[2], we[2], bb[1])
    a = _prop_kernel(t, bsrc, bdst, cnts, zrows)
    t = _round(a, t, dinv, wn[3], we[3], bb[2])
    a = _prop_kernel(t, bsrc, bdst, cnts, zrows)
    nodes_p, edges_p = _final(a, t, dinv, bb[3])

    return edges_p[:N], nodes_p[:N]


# speedup vs baseline: 2.4733x; 2.4563x over previous
import jax
import jax.numpy as jnp
from jax import lax
from jax.experimental import pallas as pl
from jax.experimental.pallas import tpu as pltpu
from jax.experimental.pallas import tpu_sc as plsc

f32 = jnp.float32
i32 = jnp.int32

N = 10000
NP = 10240
DUMMY = N
E = 320000
D = 128
CH = 128
CPT = 160
NBUF = 4
NTILES = 16
CHUNKS = NTILES * CPT
EP = CHUNKS * CH
RPT = NP // NTILES

DE = (94, 72, 50, 16)
BT = 1024


def _sc_mesh():
    return plsc.VectorSubcoreMesh(
        core_axis_name="c", subcore_axis_name="s", num_cores=2, num_subcores=16)


def _deg_body(dstc, out, idx_v, acc_v):
    c = lax.axis_index("c")
    s = lax.axis_index("s")
    w = c * NTILES + s
    zero16 = jnp.zeros((16,), f32)
    ones16 = jnp.ones((16,), f32)

    @pl.loop(0, NP // 16)
    def _(i):
        acc_v[pl.ds(i * 16, 16)] = zero16

    dpt = CHUNKS // 32
    pltpu.sync_copy(dstc.at[pl.ds(w * dpt, dpt)], idx_v)

    @pl.loop(0, dpt)
    def _(j):
        row = idx_v.at[j]
        for q in range(8):
            idx16 = row[pl.ds(q * 16, 16)]
            plsc.addupdate_scatter(acc_v, [idx16], ones16)

    @pl.when(c == 0)
    def _():
        pltpu.sync_copy(acc_v, out.at[0].at[s])

    @pl.when(c == 1)
    def _():
        pltpu.sync_copy(acc_v, out.at[1].at[s])


_deg_kernel = pl.kernel(
    _deg_body,
    out_type=jax.ShapeDtypeStruct((2, NTILES, NP), f32),
    mesh=_sc_mesh(),
    compiler_params=pltpu.CompilerParams(needs_layout_passes=False),
    scratch_types=[
        pltpu.VMEM((CHUNKS // 32, CH), i32),
        pltpu.VMEM((NP,), f32),
    ],
)


NB = 4
BR = NP // NB
RW = 32
RCH = CHUNKS // RW
EPT = RCH * CH
BRA = BR + 128
ZPT = BR // NTILES


def _bucket_body(srcc, dstc, bsrc, bdst, cnts, sidx_v, didx_v, *bufs):
    c = lax.axis_index("c")
    s = lax.axis_index("s")
    w = c * NTILES + s
    sbufs = bufs[:NB]
    dbufs = bufs[NB:2 * NB]
    cbuf_v = bufs[2 * NB]
    pltpu.sync_copy(srcc.at[pl.ds(w * RCH, RCH)], sidx_v)
    pltpu.sync_copy(dstc.at[pl.ds(w * RCH, RCH)], didx_v)

    dummy16 = jnp.full((16,), DUMMY, i32)
    lane16 = lax.iota(i32, 16)

    @pl.loop(0, EPT // 16)
    def _(i):
        trash16 = BR + lax.rem(i, 8) * 16 + lane16
        for b in range(NB):
            sbufs[b][pl.ds(i * 16, 16)] = dummy16
            dbufs[b][pl.ds(i * 16, 16)] = trash16

    def chunk(j, offs):
        srow = sidx_v.at[j]
        drow = didx_v.at[j]
        for q in range(8):
            sv = srow[pl.ds(q * 16, 16)]
            dv = drow[pl.ds(q * 16, 16)]
            new = []
            for b in range(NB):
                m = (dv >= b * BR) & (dv < (b + 1) * BR)
                cs = plsc.cumsum(jnp.where(m, 1, 0).astype(i32))
                pos = offs[b] + cs - 1
                plsc.store_scatter(sbufs[b], [pos], sv, mask=m)
                plsc.store_scatter(dbufs[b], [pos], dv - b * BR, mask=m)
                new.append(offs[b] + jnp.max(cs))
            offs = tuple(new)
        return offs

    z = jnp.int32(0)
    offs = lax.fori_loop(0, RCH, chunk, (z,) * NB)
    cv = jnp.zeros((16,), i32)
    lanes = lax.iota(i32, 16)
    for b in range(NB):
        cv = jnp.where(lanes == b, offs[b], cv)
    cbuf_v[...] = cv
    for b in range(NB):
        pltpu.sync_copy(sbufs[b], bsrc.at[w, b])
        pltpu.sync_copy(dbufs[b], bdst.at[w, b])
    pltpu.sync_copy(cbuf_v, cnts.at[w])


_bucket_kernel = pl.kernel(
    _bucket_body,
    out_type=(
        jax.ShapeDtypeStruct((RW, NB, EPT), i32),
        jax.ShapeDtypeStruct((RW, NB, EPT), i32),
        jax.ShapeDtypeStruct((RW, 16), i32),
    ),
    mesh=_sc_mesh(),
    compiler_params=pltpu.CompilerParams(needs_layout_passes=False),
    scratch_types=(
        [pltpu.VMEM((RCH, CH), i32), pltpu.VMEM((RCH, CH), i32)]
        + [pltpu.VMEM((EPT,), i32) for _ in range(2 * NB)]
        + [pltpu.VMEM((16,), i32)]
    ),
)


KPF = 3
SLOTS = 4
RCHP = RCH


def _prop_body(tab, bsrc, bdst, cnts, zrows, out, sidx_v, didx_v, didx2_v,
               rows_v, cnt_vm, acc_sh, sem_g, sem_s):
    c = lax.axis_index("c")
    s = lax.axis_index("s")
    pltpu.sync_copy(cnts, cnt_vm)
    lanes = lax.iota(i32, 16)

    def run(tab_h, out_h):
        for b in range(NB):
            pltpu.sync_copy(zrows.at[pl.ds(s * ZPT, ZPT)],
                            acc_sh.at[pl.ds(s * ZPT, ZPT)])
            plsc.subcore_barrier()
            for r in range(2):
                w = s * 2 + r
                pltpu.sync_copy(bsrc.at[w, b], sidx_v)
                pltpu.sync_copy(bdst.at[w, b], didx_v)
                cv = cnt_vm[w]
                cnt = jnp.max(jnp.where(lanes == b, cv, 0))
                nch = lax.div(cnt + (CH - 1), CH)

                def gather(j, t):
                    pltpu.async_copy(
                        tab_h.at[sidx_v.at[pl.ds(j * CH, CH)]],
                        rows_v.at[t], sem_g.at[t])

                for t in range(KPF):
                    @pl.when(t < nch)
                    def _():
                        gather(t, t)

                def group(g, _):
                    for t in range(SLOTS):
                        j = g * SLOTS + t

                        @pl.when(j < nch)
                        def _():
                            pltpu.make_async_copy(
                                tab_h.at[pl.ds(0, CH)], rows_v.at[t],
                                sem_g.at[t]).wait()
                            for q in range(CH // 16):
                                didx2_v[0, pl.ds(q * 16, 16)] = (
                                    didx_v[pl.ds(j * CH + q * 16, 16)])
                            t2 = (t + KPF) % SLOTS

                            @pl.when(j + KPF < nch)
                            def _():
                                gather(j + KPF, t2)
                            pltpu.sync_copy(rows_v.at[t],
                                            acc_sh.at[didx2_v.at[0]],
                                            add=True)
                    return 0

                lax.fori_loop(0, lax.div(nch + (SLOTS - 1), SLOTS), group, 0)
            plsc.subcore_barrier()
            pltpu.sync_copy(acc_sh.at[pl.ds(s * ZPT, ZPT)],
                            out_h.at[pl.ds(b * BR + s * ZPT, ZPT)])
            plsc.subcore_barrier()

    @pl.when(c == 0)
    def _():
        run(tab.at[0], out.at[0])

    @pl.when(c == 1)
    def _():
        run(tab.at[1], out.at[1])


_prop_kernel = pl.kernel(
    _prop_body,
    out_type=jax.ShapeDtypeStruct((2, NP, D), f32),
    mesh=_sc_mesh(),
    compiler_params=pltpu.CompilerParams(needs_layout_passes=False),
    scratch_types=[
        pltpu.VMEM((EPT,), i32),
        pltpu.VMEM((EPT,), i32),
        pltpu.VMEM((1, CH), i32),
        pltpu.VMEM((SLOTS, CH, D), f32),
        pltpu.VMEM((RW, 16), i32),
        pltpu.VMEM_SHARED((BRA, D), f32),
        pltpu.SemaphoreType.DMA((SLOTS,)),
        pltpu.SemaphoreType.DMA((SLOTS,)),
    ],
)


def _round0_body(x_ref, deg_ref, wn_ref, we_ref, out_ref, dinv_ref):
    i = pl.program_id(0)
    deg = jnp.sum(deg_ref[...], axis=(0, 1)) + 1.0
    rows = i * BT + lax.broadcasted_iota(i32, (BT,), 0)
    dinv = jnp.where(rows < N, lax.rsqrt(deg), 0.0)
    dinv_ref[...] = dinv
    xv = x_ref[...]
    un = jnp.dot(xv, wn_ref[...], preferred_element_type=f32,
                 precision=lax.Precision.HIGHEST)
    ue = jnp.dot(xv, we_ref[...], preferred_element_type=f32,
                 precision=lax.Precision.HIGHEST)
    out_ref[0] = dinv[:, None] * un
    out_ref[1] = dinv[:, None] * ue


_round0 = pl.pallas_call(
    _round0_body,
    grid=(NP // BT,),
    in_specs=[
        pl.BlockSpec((BT, D), lambda i: (i, 0)),
        pl.BlockSpec((2, NTILES, BT), lambda i: (0, 0, i)),
        pl.BlockSpec((D, D), lambda i: (0, 0)),
        pl.BlockSpec((D, D), lambda i: (0, 0)),
    ],
    out_specs=[
        pl.BlockSpec((2, BT, D), lambda i: (0, i, 0)),
        pl.BlockSpec((BT,), lambda i: (i,)),
    ],
    out_shape=[
        jax.ShapeDtypeStruct((2, NP, D), f32),
        jax.ShapeDtypeStruct((NP,), f32),
    ],
)


def _round_body(acc_ref, tab_ref, dinv_ref, wn_ref, we_ref, b_ref, out_ref):
    dinv = dinv_ref[...][:, None]
    pre_n = dinv * (acc_ref[0] + tab_ref[0]) + b_ref[0]
    pre_e = dinv * (acc_ref[1] + tab_ref[1]) + b_ref[1]
    act_n = jnp.maximum(pre_n, 0.0)
    act_e = jnp.maximum(pre_e, 0.0)
    un = jnp.dot(act_n, wn_ref[...], preferred_element_type=f32,
                 precision=lax.Precision.HIGHEST)
    ue = jnp.dot(act_e, we_ref[...], preferred_element_type=f32,
                 precision=lax.Precision.HIGHEST)
    out_ref[0] = dinv * un
    out_ref[1] = dinv * ue


_round = pl.pallas_call(
    _round_body,
    grid=(NP // BT,),
    in_specs=[
        pl.BlockSpec((2, BT, D), lambda i: (0, i, 0)),
        pl.BlockSpec((2, BT, D), lambda i: (0, i, 0)),
        pl.BlockSpec((BT,), lambda i: (i,)),
        pl.BlockSpec((D, D), lambda i: (0, 0)),
        pl.BlockSpec((D, D), lambda i: (0, 0)),
        pl.BlockSpec((2, 1, D), lambda i: (0, 0, 0)),
    ],
    out_specs=pl.BlockSpec((2, BT, D), lambda i: (0, i, 0)),
    out_shape=jax.ShapeDtypeStruct((2, NP, D), f32),
)


def _final_body(acc_ref, tab_ref, dinv_ref, b_ref, outn_ref, oute_ref):
    dinv = dinv_ref[...][:, None]
    pre_n = dinv * (acc_ref[0] + tab_ref[0]) + b_ref[0]
    pre_e = dinv * (acc_ref[1] + tab_ref[1]) + b_ref[1]
    outn_ref[...] = jax.nn.sigmoid(pre_n)
    oute_ref[...] = jax.nn.sigmoid(pre_e[:, :16])


_final = pl.pallas_call(
    _final_body,
    grid=(NP // BT,),
    in_specs=[
        pl.BlockSpec((2, BT, D), lambda i: (0, i, 0)),
        pl.BlockSpec((2, BT, D), lambda i: (0, i, 0)),
        pl.BlockSpec((BT,), lambda i: (i,)),
        pl.BlockSpec((2, 1, D), lambda i: (0, 0, 0)),
    ],
    out_specs=[
        pl.BlockSpec((BT, D), lambda i: (i, 0)),
        pl.BlockSpec((BT, 16), lambda i: (i, 0)),
    ],
    out_shape=[
        jax.ShapeDtypeStruct((NP, D), f32),
        jax.ShapeDtypeStruct((NP, 16), f32),
    ],
)


def _pad_w(W):
    din, dout = W.shape
    return jnp.zeros((D, D), f32).at[:din, :dout].set(W)


def _pad_b2(bn, be):
    b = jnp.zeros((2, 1, D), f32)
    return b.at[0, 0, :].set(bn).at[1, 0, :be.shape[0]].set(be)


def kernel(x, edge_index, W1e, b1e, W2e, b2e, W3e, b3e, W4e, b4e,
           W1n, b1n, W2n, b2n, W3n, b3n, W4n, b4n):
    src = edge_index[0].astype(i32)
    dst = edge_index[1].astype(i32)
    pad = jnp.full((EP - E,), DUMMY, i32)
    srcc = jnp.concatenate([src, pad]).reshape(CHUNKS, CH)
    dstc = jnp.concatenate([dst, pad]).reshape(CHUNKS, CH)
    xp = jnp.pad(x, ((0, NP - N), (0, 0)))
    zrows = jnp.zeros((NP, D), f32)

    wn = [_pad_w(W1n), _pad_w(W2n), _pad_w(W3n), _pad_w(W4n)]
    we = [_pad_w(W1e), _pad_w(W2e), _pad_w(W3e), _pad_w(W4e)]
    bb = [_pad_b2(b1n, b1e), _pad_b2(b2n, b2e), _pad_b2(b3n, b3e),
          _pad_b2(b4n, b4e)]

    deg_parts = _deg_kernel(dstc)
    bsrc, bdst, cnts = _bucket_kernel(srcc, dstc)
    t, dinv = _round0(xp, deg_parts, wn[0], we[0])
    a = _prop_kernel(t, bsrc, bdst, cnts, zrows)
    t = _round(a, t, dinv, wn[1], we[1], bb[0])
    a = _prop_kernel(t, bsrc, bdst, cnts, zrows)
    t = _round(a, t, dinv, wn[2], we[2], bb[1])
    a = _prop_kernel(t, bsrc, bdst, cnts, zrows)
    t = _round(a, t, dinv, wn[3], we[3], bb[2])
    a = _prop_kernel(t, bsrc, bdst, cnts, zrows)
    nodes_p, edges_p = _final(a, t, dinv, bb[3])

    return edges_p[:N], nodes_p[:N]
